# baseline (device time: 274052 ns/iter reference)
import jax
import jax.numpy as jnp
from jax import lax
from jax.experimental import pallas as pl
from jax.experimental.pallas import tpu as pltpu

G = 8
HC = 128
LAST_T = 2 * (G - 1) - 1

_MESH = pl.DeviceIdType.MESH

_INV = (0, 7, 6, 1, 2, 5, 4, 3)
_NEXT = (3, 0, 1, 4, 7, 2, 5, 6)
_PREV = (1, 2, 5, 0, 3, 6, 7, 4)


def _lut(idx, table):
    v = jnp.int32(table[7])
    for i in range(6, -1, -1):
        v = jnp.where(idx == i, jnp.int32(table[i]), v)
    return v


def _ar_body(p_ref, out_ref, acc_ref, commg_ref, stage_ref, commz_ref,
             sendg_sem, recvg_sem, z_send_sem, z_recv_sem, credit_sem):
    my = lax.axis_index("i")
    p = my % G
    z = my // G
    r = _lut(p, _INV)
    right = z * G + _lut(p, _NEXT)
    left = z * G + _lut(p, _PREV)
    zp1 = my ^ 8
    zp2 = my ^ 16

    barrier = pltpu.get_barrier_semaphore()
    for nbr in (left, right, zp1, zp2):
        pl.semaphore_signal(barrier, inc=1, device_id=(nbr,),
                            device_id_type=_MESH)
    pl.semaphore_wait(barrier, 4)

    bf = out_ref.dtype
    f32 = jnp.float32

    def ring_rdma(slot, src, dst, dev):
        return pltpu.make_async_remote_copy(
            src_ref=src, dst_ref=dst,
            send_sem=sendg_sem.at[slot], recv_sem=recvg_sem.at[slot],
            device_id=(dev,), device_id_type=_MESH,
        )

    def z_rdma(i, src, dst, dev):
        return pltpu.make_async_remote_copy(
            src_ref=src, dst_ref=dst,
            send_sem=z_send_sem.at[i], recv_sem=z_recv_sem.at[i],
            device_id=(dev,), device_id_type=_MESH,
        )

    def credit_to(slot, dev):
        pl.semaphore_signal(credit_sem.at[slot], inc=1, device_id=(dev,),
                            device_id_type=_MESH)

    rs = []
    for k in range(G - 1):
        sR = k % 2
        sL = 2 + k % 2
        s_idxR = (r - k) % G
        s_idxL = (r + k) % G
        if k >= 2:
            pl.semaphore_wait(credit_sem.at[sR], 1)
            pl.semaphore_wait(credit_sem.at[sL], 1)
            rs[k - 2][0].wait_send()
            rs[k - 2][1].wait_send()
        if k == 0:
            stage_ref[sR] = p_ref[s_idxR, 0:HC].astype(bf)
            stage_ref[sL] = p_ref[s_idxL, HC:2 * HC].astype(bf)
        else:
            pR = (k - 1) % 2
            pL = 2 + (k - 1) % 2
            stage_ref[sR] = (
                p_ref[s_idxR, 0:HC] + commg_ref[pR].astype(f32)
            ).astype(bf)
            stage_ref[sL] = (
                p_ref[s_idxL, HC:2 * HC] + commg_ref[pL].astype(f32)
            ).astype(bf)
            credit_to(pR, left)
            credit_to(pL, right)
        rdmaR = ring_rdma(sR, stage_ref.at[sR], commg_ref.at[sR], right)
        rdmaL = ring_rdma(sL, stage_ref.at[sL], commg_ref.at[sL], left)
        rdmaR.start()
        rdmaL.start()
        rs.append((rdmaR, rdmaL))
        rdmaR.wait_recv()
        rdmaL.wait_recv()

    oR = (r + 1) % G
    oL = (r - 1) % G
    acc_ref[0] = p_ref[oR, 0:HC] + commg_ref[0].astype(f32)
    acc_ref[1] = p_ref[oL, HC:2 * HC] + commg_ref[2].astype(f32)
    credit_to(0, left)
    credit_to(2, right)

    zb0 = z % 2
    zb1 = z // 2
    hh = zb0 * 64
    qq = hh + zb1 * 32
    sq = hh + (1 - zb1) * 32
    qk = zb1 * 32
    qs = (1 - zb1) * 32

    pend = []

    rs[6][0].wait_send()
    rs[6][1].wait_send()
    stage_ref[0, 0:64] = acc_ref[0, pl.ds((1 - zb0) * 64, 64)].astype(bf)
    stage_ref[2, 0:64] = acc_ref[1, pl.ds((1 - zb0) * 64, 64)].astype(bf)
    ex0R = z_rdma(0, stage_ref.at[0, pl.ds(0, 64)], commz_ref.at[0], zp1)
    ex0L = z_rdma(1, stage_ref.at[2, pl.ds(0, 64)], commz_ref.at[1], zp1)
    ex0R.start()
    ex0L.start()
    ex0R.wait_recv()
    ex0L.wait_recv()
    pend += [ex0R, ex0L]

    rs[5][0].wait_send()
    rs[5][1].wait_send()
    stage_ref[1, 0:32] = (
        acc_ref[0, pl.ds(sq, 32)] + commz_ref[0, pl.ds(qs, 32)].astype(f32)
    ).astype(bf)
    stage_ref[3, 0:32] = (
        acc_ref[1, pl.ds(sq, 32)] + commz_ref[1, pl.ds(qs, 32)].astype(f32)
    ).astype(bf)
    ex1R = z_rdma(2, stage_ref.at[1, pl.ds(0, 32)],
                  commz_ref.at[2, pl.ds(0, 32)], zp2)
    ex1L = z_rdma(3, stage_ref.at[3, pl.ds(0, 32)],
                  commz_ref.at[3, pl.ds(0, 32)], zp2)
    ex1R.start()
    ex1L.start()
    acc_ref[0, pl.ds(qq, 32)] = (
        acc_ref[0, pl.ds(qq, 32)] + commz_ref[0, pl.ds(qk, 32)].astype(f32)
    )
    acc_ref[1, pl.ds(qq, 32)] = (
        acc_ref[1, pl.ds(qq, 32)] + commz_ref[1, pl.ds(qk, 32)].astype(f32)
    )
    ex1R.wait_recv()
    ex1L.wait_recv()
    out_ref[oR, pl.ds(qq, 32)] = (
        acc_ref[0, pl.ds(qq, 32)] + commz_ref[2, pl.ds(0, 32)].astype(f32)
    ).astype(bf)
    out_ref[oL, pl.ds(HC + qq, 32)] = (
        acc_ref[1, pl.ds(qq, 32)] + commz_ref[3, pl.ds(0, 32)].astype(f32)
    ).astype(bf)
    pend += [ex1R, ex1L]

    ag2R = z_rdma(4, out_ref.at[oR, pl.ds(qq, 32)],
                  out_ref.at[oR, pl.ds(qq, 32)], zp2)
    ag2L = z_rdma(5, out_ref.at[oL, pl.ds(HC + qq, 32)],
                  out_ref.at[oL, pl.ds(HC + qq, 32)], zp2)
    ag2R.start()
    ag2L.start()
    ag2R.wait_recv()
    ag2L.wait_recv()
    pend += [ag2R, ag2L]

    ag3R = z_rdma(6, out_ref.at[oR, pl.ds(hh, 64)],
                  out_ref.at[oR, pl.ds(hh, 64)], zp1)
    ag3L = z_rdma(7, out_ref.at[oL, pl.ds(HC + hh, 64)],
                  out_ref.at[oL, pl.ds(HC + hh, 64)], zp1)
    ag3R.start()
    ag3L.start()
    ag3R.wait_recv()
    ag3L.wait_recv()
    pend += [ag3R, ag3L]

    ag = []
    for g in range(G - 1):
        t = (G - 1) + g
        sR = t % 2
        sL = 2 + t % 2
        s_idxR = (r + 1 - g) % G
        s_idxL = (r - 1 + g) % G
        pl.semaphore_wait(credit_sem.at[sR], 1)
        pl.semaphore_wait(credit_sem.at[sL], 1)
        if g >= 1:
            ag[g - 1][0].wait_recv()
            ag[g - 1][1].wait_recv()
            if g <= 5:
                credit_to((t - 1) % 2, left)
                credit_to(2 + (t - 1) % 2, right)
        agR = ring_rdma(sR, out_ref.at[s_idxR, pl.ds(0, HC)],
                        out_ref.at[s_idxR, pl.ds(0, HC)], right)
        agL = ring_rdma(sL, out_ref.at[s_idxL, pl.ds(HC, HC)],
                        out_ref.at[s_idxL, pl.ds(HC, HC)], left)
        agR.start()
        agL.start()
        ag.append((agR, agL))
    ag[6][0].wait_recv()
    ag[6][1].wait_recv()

    for r in pend:
        r.wait_send()
    for pair in ag:
        pair[0].wait_send()
        pair[1].wait_send()


def _ring_allreduce(partial, cid):
    B, S, D = partial.shape
    rows = B * S
    c = rows // G
    p = partial.reshape(G, c, D)
    out = pl.pallas_call(
        _ar_body,
        out_shape=jax.ShapeDtypeStruct((G, c, D), jnp.bfloat16),
        in_specs=[pl.BlockSpec(memory_space=pltpu.VMEM)],
        out_specs=pl.BlockSpec(memory_space=pltpu.VMEM),
        scratch_shapes=[
            pltpu.VMEM((2, HC, D), jnp.float32),
            pltpu.VMEM((4, HC, D), jnp.bfloat16),
            pltpu.VMEM((4, HC, D), jnp.bfloat16),
            pltpu.VMEM((4, 64, D), jnp.bfloat16),
            pltpu.SemaphoreType.DMA((4,)),
            pltpu.SemaphoreType.DMA((4,)),
            pltpu.SemaphoreType.DMA((8,)),
            pltpu.SemaphoreType.DMA((8,)),
            pltpu.SemaphoreType.REGULAR((4,)),
        ],
        compiler_params=pltpu.CompilerParams(collective_id=cid),
    )(p)
    return out.reshape(B, S, D)


def _attn_body(q_ref, k_ref, v_ref, o_ref):
    f32 = jnp.float32
    q = q_ref[0]
    k = k_ref[0]
    v = v_ref[0]
    s = lax.dot_general(
        q, k, (((1,), (1,)), ((), ())), preferred_element_type=f32
    ) * 0.08838834764831843
    m = jnp.max(s, axis=-1, keepdims=True)
    e = jnp.exp(s - m)
    l = jnp.sum(e, axis=-1, keepdims=True)
    p = (e / l).astype(o_ref.dtype)
    o_ref[0] = lax.dot_general(
        p, v, (((1,), (0,)), ((), ())), preferred_element_type=f32
    ).astype(o_ref.dtype)


def _flash_attention(Q, K, V):
    BH, S, Dh = Q.shape
    spec = pl.BlockSpec((1, S, Dh), lambda i: (i, 0, 0))
    return pl.pallas_call(
        _attn_body,
        grid=(BH,),
        in_specs=[spec, spec, spec],
        out_specs=spec,
        out_shape=jax.ShapeDtypeStruct((BH, S, Dh), Q.dtype),
    )(Q, K, V)


def kernel(x, Wq, Wk, Wv, Wo, t_emb, W_mod, W_ff1, W_ff2):
    f32 = jnp.float32
    bf16 = jnp.bfloat16
    B, S, D = x.shape
    Dh = 128
    H = Wq.shape[1] // Dh

    mod = t_emb @ W_mod
    sa, sha, ga, sm, shm, gm = jnp.split(mod, 6, axis=-1)

    def ln(h):
        m = h.mean(axis=-1, keepdims=True)
        v = h.var(axis=-1, keepdims=True)
        return (h - m) * lax.rsqrt(v + 1e-5)

    x0 = x
    xm = (ln(x0) * (1.0 + sa[:, None, :]) + sha[:, None, :]).astype(bf16)

    def heads(w):
        y = (xm @ w.astype(bf16)).reshape(B, S, H, Dh)
        return y.transpose(0, 2, 1, 3).reshape(B * H, S, Dh)

    O = _flash_attention(heads(Wq), heads(Wk), heads(Wv))
    O = O.reshape(B, H, S, Dh).transpose(0, 2, 1, 3).reshape(B, S, H * Dh)
    attn_partial = (O @ Wo.astype(bf16)).astype(f32)

    attn_sum = _ring_allreduce(attn_partial, cid=0).astype(f32)
    x1 = x0 + ga[:, None, :] * attn_sum

    xm2 = (ln(x1) * (1.0 + sm[:, None, :]) + shm[:, None, :]).astype(bf16)
    h = (xm2 @ W_ff1.astype(bf16)).astype(f32)
    h = h * jax.nn.sigmoid(h)
    ff_partial = (h.astype(bf16) @ W_ff2.astype(bf16)).astype(f32)

    ff_sum = _ring_allreduce(ff_partial, cid=1).astype(f32)
    return x1 + gm[:, None, :] * ff_sum


# device time: 269503 ns/iter; 1.0169x vs baseline; 1.0169x over previous
import jax
import jax.numpy as jnp
from jax import lax
from jax.experimental import pallas as pl
from jax.experimental.pallas import tpu as pltpu

G = 8
HC = 128
LAST_T = 2 * (G - 1) - 1

_MESH = pl.DeviceIdType.MESH

_INV = (0, 7, 6, 1, 2, 5, 4, 3)
_NEXT = (3, 0, 1, 4, 7, 2, 5, 6)
_PREV = (1, 2, 5, 0, 3, 6, 7, 4)


def _lut(idx, table):
    v = jnp.int32(table[7])
    for i in range(6, -1, -1):
        v = jnp.where(idx == i, jnp.int32(table[i]), v)
    return v


def _ar_body(p_ref, out_ref, acc_ref, commg_ref, stage_ref, commz_ref,
             sendg_sem, recvg_sem, z_send_sem, z_recv_sem, credit_sem):
    my = lax.axis_index("i")
    p = my % G
    z = my // G
    r = _lut(p, _INV)
    right = z * G + _lut(p, _NEXT)
    left = z * G + _lut(p, _PREV)
    zp1 = my ^ 8
    zp2 = my ^ 16

    barrier = pltpu.get_barrier_semaphore()
    for nbr in (left, right, zp1, zp2):
        pl.semaphore_signal(barrier, inc=1, device_id=(nbr,),
                            device_id_type=_MESH)
    pl.semaphore_wait(barrier, 4)

    bf = out_ref.dtype
    f32 = jnp.float32

    def ring_rdma(slot, src, dst, dev):
        return pltpu.make_async_remote_copy(
            src_ref=src, dst_ref=dst,
            send_sem=sendg_sem.at[slot], recv_sem=recvg_sem.at[slot],
            device_id=(dev,), device_id_type=_MESH,
        )

    def z_rdma(i, src, dst, dev):
        return pltpu.make_async_remote_copy(
            src_ref=src, dst_ref=dst,
            send_sem=z_send_sem.at[i], recv_sem=z_recv_sem.at[i],
            device_id=(dev,), device_id_type=_MESH,
        )

    def credit_to(slot, dev):
        pl.semaphore_signal(credit_sem.at[slot], inc=1, device_id=(dev,),
                            device_id_type=_MESH)

    rs = []
    for k in range(G - 1):
        sR = k % 2
        sL = 2 + k % 2
        s_idxR = (r - k) % G
        s_idxL = (r + k) % G
        if k >= 2:
            pl.semaphore_wait(credit_sem.at[sR], 1)
            pl.semaphore_wait(credit_sem.at[sL], 1)
            rs[k - 2][0].wait_send()
            rs[k - 2][1].wait_send()
        if k == 0:
            stage_ref[sR] = p_ref[s_idxR, 0:HC].astype(bf)
            stage_ref[sL] = p_ref[s_idxL, HC:2 * HC].astype(bf)
        else:
            pR = (k - 1) % 2
            pL = 2 + (k - 1) % 2
            stage_ref[sR] = (
                p_ref[s_idxR, 0:HC] + commg_ref[pR].astype(f32)
            ).astype(bf)
            stage_ref[sL] = (
                p_ref[s_idxL, HC:2 * HC] + commg_ref[pL].astype(f32)
            ).astype(bf)
            credit_to(pR, left)
            credit_to(pL, right)
        rdmaR = ring_rdma(sR, stage_ref.at[sR], commg_ref.at[sR], right)
        rdmaL = ring_rdma(sL, stage_ref.at[sL], commg_ref.at[sL], left)
        rdmaR.start()
        rdmaL.start()
        rs.append((rdmaR, rdmaL))
        rdmaR.wait_recv()
        rdmaL.wait_recv()

    oR = (r + 1) % G
    oL = (r - 1) % G
    acc_ref[0] = p_ref[oR, 0:HC] + commg_ref[0].astype(f32)
    acc_ref[1] = p_ref[oL, HC:2 * HC] + commg_ref[2].astype(f32)
    credit_to(0, left)
    credit_to(2, right)

    zb0 = z % 2
    zb1 = z // 2
    hh = zb0 * 64
    qq = hh + zb1 * 32
    sq = hh + (1 - zb1) * 32
    qk = zb1 * 32
    qs = (1 - zb1) * 32

    pend = []

    rs[6][0].wait_send()
    rs[6][1].wait_send()
    stage_ref[0, 0:64] = acc_ref[0, pl.ds((1 - zb0) * 64, 64)].astype(bf)
    stage_ref[2, 0:64] = acc_ref[1, pl.ds((1 - zb0) * 64, 64)].astype(bf)
    ex0R = z_rdma(0, stage_ref.at[0, pl.ds(0, 64)], commz_ref.at[0], zp1)
    ex0L = z_rdma(1, stage_ref.at[2, pl.ds(0, 64)], commz_ref.at[1], zp1)
    ex0R.start()
    ex0L.start()
    ex0R.wait_recv()
    ex0L.wait_recv()
    pend += [ex0R, ex0L]

    rs[5][0].wait_send()
    rs[5][1].wait_send()
    stage_ref[1, 0:32] = (
        acc_ref[0, pl.ds(sq, 32)] + commz_ref[0, pl.ds(qs, 32)].astype(f32)
    ).astype(bf)
    stage_ref[3, 0:32] = (
        acc_ref[1, pl.ds(sq, 32)] + commz_ref[1, pl.ds(qs, 32)].astype(f32)
    ).astype(bf)
    ex1R = z_rdma(2, stage_ref.at[1, pl.ds(0, 32)],
                  commz_ref.at[2, pl.ds(0, 32)], zp2)
    ex1L = z_rdma(3, stage_ref.at[3, pl.ds(0, 32)],
                  commz_ref.at[3, pl.ds(0, 32)], zp2)
    ex1R.start()
    ex1L.start()
    acc_ref[0, pl.ds(qq, 32)] = (
        acc_ref[0, pl.ds(qq, 32)] + commz_ref[0, pl.ds(qk, 32)].astype(f32)
    )
    acc_ref[1, pl.ds(qq, 32)] = (
        acc_ref[1, pl.ds(qq, 32)] + commz_ref[1, pl.ds(qk, 32)].astype(f32)
    )
    ex1R.wait_recv()
    ex1L.wait_recv()
    out_ref[oR, pl.ds(qq, 32)] = (
        acc_ref[0, pl.ds(qq, 32)] + commz_ref[2, pl.ds(0, 32)].astype(f32)
    ).astype(bf)
    out_ref[oL, pl.ds(HC + qq, 32)] = (
        acc_ref[1, pl.ds(qq, 32)] + commz_ref[3, pl.ds(0, 32)].astype(f32)
    ).astype(bf)
    pend += [ex1R, ex1L]

    ag2R = z_rdma(4, out_ref.at[oR, pl.ds(qq, 32)],
                  out_ref.at[oR, pl.ds(qq, 32)], zp2)
    ag2L = z_rdma(5, out_ref.at[oL, pl.ds(HC + qq, 32)],
                  out_ref.at[oL, pl.ds(HC + qq, 32)], zp2)
    ag2R.start()
    ag2L.start()
    ag2R.wait_recv()
    ag2L.wait_recv()
    pend += [ag2R, ag2L]

    ag3R = z_rdma(6, out_ref.at[oR, pl.ds(hh, 64)],
                  out_ref.at[oR, pl.ds(hh, 64)], zp1)
    ag3L = z_rdma(7, out_ref.at[oL, pl.ds(HC + hh, 64)],
                  out_ref.at[oL, pl.ds(HC + hh, 64)], zp1)
    ag3R.start()
    ag3L.start()
    ag3R.wait_recv()
    ag3L.wait_recv()
    pend += [ag3R, ag3L]

    ag = []
    for g in range(G - 1):
        t = (G - 1) + g
        sR = t % 2
        sL = 2 + t % 2
        s_idxR = (r + 1 - g) % G
        s_idxL = (r - 1 + g) % G
        pl.semaphore_wait(credit_sem.at[sR], 1)
        pl.semaphore_wait(credit_sem.at[sL], 1)
        if g >= 1:
            ag[g - 1][0].wait_recv()
            ag[g - 1][1].wait_recv()
            if g <= 5:
                credit_to((t - 1) % 2, left)
                credit_to(2 + (t - 1) % 2, right)
        agR = ring_rdma(sR, out_ref.at[s_idxR, pl.ds(0, HC)],
                        out_ref.at[s_idxR, pl.ds(0, HC)], right)
        agL = ring_rdma(sL, out_ref.at[s_idxL, pl.ds(HC, HC)],
                        out_ref.at[s_idxL, pl.ds(HC, HC)], left)
        agR.start()
        agL.start()
        ag.append((agR, agL))
    ag[6][0].wait_recv()
    ag[6][1].wait_recv()

    for r in pend:
        r.wait_send()
    for pair in ag:
        pair[0].wait_send()
        pair[1].wait_send()


def _ring_allreduce(partial, cid):
    B, S, D = partial.shape
    rows = B * S
    c = rows // G
    p = partial.reshape(G, c, D)
    out = pl.pallas_call(
        _ar_body,
        out_shape=jax.ShapeDtypeStruct((G, c, D), jnp.bfloat16),
        in_specs=[pl.BlockSpec(memory_space=pltpu.VMEM)],
        out_specs=pl.BlockSpec(memory_space=pltpu.VMEM),
        scratch_shapes=[
            pltpu.VMEM((2, HC, D), jnp.float32),
            pltpu.VMEM((4, HC, D), jnp.bfloat16),
            pltpu.VMEM((4, HC, D), jnp.bfloat16),
            pltpu.VMEM((4, 64, D), jnp.bfloat16),
            pltpu.SemaphoreType.DMA((4,)),
            pltpu.SemaphoreType.DMA((4,)),
            pltpu.SemaphoreType.DMA((8,)),
            pltpu.SemaphoreType.DMA((8,)),
            pltpu.SemaphoreType.REGULAR((4,)),
        ],
        compiler_params=pltpu.CompilerParams(collective_id=cid),
    )(p)
    return out.reshape(B, S, D)


def _attn_body(q_ref, k_ref, v_ref, o_ref):
    f32 = jnp.float32
    q = q_ref[0]
    k = k_ref[0]
    v = v_ref[0]
    s = lax.dot_general(
        q, k, (((1,), (1,)), ((), ())), preferred_element_type=f32
    ) * 0.08838834764831843
    m = jnp.max(s, axis=-1, keepdims=True)
    e = jnp.exp(s - m)
    l = jnp.sum(e, axis=-1, keepdims=True)
    o = lax.dot_general(
        e.astype(o_ref.dtype), v, (((1,), (0,)), ((), ())),
        preferred_element_type=f32,
    )
    o_ref[0] = (o / l).astype(o_ref.dtype)


def _flash_attention(Q, K, V, Dh):
    B, S, HD = Q.shape
    H = HD // Dh
    spec = pl.BlockSpec((1, S, Dh), lambda b, h: (b, 0, h))
    return pl.pallas_call(
        _attn_body,
        grid=(B, H),
        in_specs=[spec, spec, spec],
        out_specs=spec,
        out_shape=jax.ShapeDtypeStruct((B, S, HD), Q.dtype),
    )(Q, K, V)


def kernel(x, Wq, Wk, Wv, Wo, t_emb, W_mod, W_ff1, W_ff2):
    f32 = jnp.float32
    bf16 = jnp.bfloat16
    B, S, D = x.shape
    Dh = 128
    H = Wq.shape[1] // Dh

    mod = t_emb @ W_mod
    sa, sha, ga, sm, shm, gm = jnp.split(mod, 6, axis=-1)

    def ln(h):
        m = h.mean(axis=-1, keepdims=True)
        v = h.var(axis=-1, keepdims=True)
        return (h - m) * lax.rsqrt(v + 1e-5)

    x0 = x
    xm = (ln(x0) * (1.0 + sa[:, None, :]) + sha[:, None, :]).astype(bf16)

    O = _flash_attention(
        xm @ Wq.astype(bf16), xm @ Wk.astype(bf16), xm @ Wv.astype(bf16), Dh
    )
    attn_partial = (O @ Wo.astype(bf16)).astype(f32)

    attn_sum = _ring_allreduce(attn_partial, cid=0).astype(f32)
    x1 = x0 + ga[:, None, :] * attn_sum

    xm2 = (ln(x1) * (1.0 + sm[:, None, :]) + shm[:, None, :]).astype(bf16)
    h = (xm2 @ W_ff1.astype(bf16)).astype(f32)
    h = h * jax.nn.sigmoid(h)
    ff_partial = (h.astype(bf16) @ W_ff2.astype(bf16)).astype(f32)

    ff_sum = _ring_allreduce(ff_partial, cid=1).astype(f32)
    return x1 + gm[:, None, :] * ff_sum


# device time: 267034 ns/iter; 1.0263x vs baseline; 1.0092x over previous
import jax
import jax.numpy as jnp
from jax import lax
from jax.experimental import pallas as pl
from jax.experimental.pallas import tpu as pltpu

G = 8
HC = 128
LAST_T = 2 * (G - 1) - 1

_MESH = pl.DeviceIdType.MESH

_INV = (0, 7, 6, 1, 2, 5, 4, 3)
_NEXT = (3, 0, 1, 4, 7, 2, 5, 6)
_PREV = (1, 2, 5, 0, 3, 6, 7, 4)


def _lut(idx, table):
    v = jnp.int32(table[7])
    for i in range(6, -1, -1):
        v = jnp.where(idx == i, jnp.int32(table[i]), v)
    return v


def _ar_core(p_ref, out_ref, acc_ref, commg_ref, stage_ref, commz_ref,
             sendg_sem, recvg_sem, z_send_sem, z_recv_sem, credit_sem):
    my = lax.axis_index("i")
    p = my % G
    z = my // G
    r = _lut(p, _INV)
    right = z * G + _lut(p, _NEXT)
    left = z * G + _lut(p, _PREV)
    zp1 = my ^ 8
    zp2 = my ^ 16

    barrier = pltpu.get_barrier_semaphore()
    for nbr in (left, right, zp1, zp2):
        pl.semaphore_signal(barrier, inc=1, device_id=(nbr,),
                            device_id_type=_MESH)
    pl.semaphore_wait(barrier, 4)

    bf = out_ref.dtype
    f32 = jnp.float32

    def ring_rdma(slot, src, dst, dev):
        return pltpu.make_async_remote_copy(
            src_ref=src, dst_ref=dst,
            send_sem=sendg_sem.at[slot], recv_sem=recvg_sem.at[slot],
            device_id=(dev,), device_id_type=_MESH,
        )

    def z_rdma(i, src, dst, dev):
        return pltpu.make_async_remote_copy(
            src_ref=src, dst_ref=dst,
            send_sem=z_send_sem.at[i], recv_sem=z_recv_sem.at[i],
            device_id=(dev,), device_id_type=_MESH,
        )

    def credit_to(slot, dev):
        pl.semaphore_signal(credit_sem.at[slot], inc=1, device_id=(dev,),
                            device_id_type=_MESH)

    rs = []
    for k in range(G - 1):
        sR = k % 2
        sL = 2 + k % 2
        s_idxR = (r - k) % G
        s_idxL = (r + k) % G
        if k >= 2:
            pl.semaphore_wait(credit_sem.at[sR], 1)
            pl.semaphore_wait(credit_sem.at[sL], 1)
            rs[k - 2][0].wait_send()
            rs[k - 2][1].wait_send()
        if k == 0:
            stage_ref[sR] = p_ref[s_idxR, 0:HC].astype(bf)
            stage_ref[sL] = p_ref[s_idxL, HC:2 * HC].astype(bf)
        else:
            pR = (k - 1) % 2
            pL = 2 + (k - 1) % 2
            stage_ref[sR] = (
                p_ref[s_idxR, 0:HC] + commg_ref[pR].astype(f32)
            ).astype(bf)
            stage_ref[sL] = (
                p_ref[s_idxL, HC:2 * HC] + commg_ref[pL].astype(f32)
            ).astype(bf)
            credit_to(pR, left)
            credit_to(pL, right)
        rdmaR = ring_rdma(sR, stage_ref.at[sR], commg_ref.at[sR], right)
        rdmaL = ring_rdma(sL, stage_ref.at[sL], commg_ref.at[sL], left)
        rdmaR.start()
        rdmaL.start()
        rs.append((rdmaR, rdmaL))
        rdmaR.wait_recv()
        rdmaL.wait_recv()

    oR = (r + 1) % G
    oL = (r - 1) % G
    acc_ref[0] = p_ref[oR, 0:HC] + commg_ref[0].astype(f32)
    acc_ref[1] = p_ref[oL, HC:2 * HC] + commg_ref[2].astype(f32)
    credit_to(0, left)
    credit_to(2, right)

    zb0 = z % 2
    zb1 = z // 2
    hh = zb0 * 64
    qq = hh + zb1 * 32
    sq = hh + (1 - zb1) * 32
    qk = zb1 * 32
    qs = (1 - zb1) * 32

    pend = []

    rs[6][0].wait_send()
    rs[6][1].wait_send()
    stage_ref[0, 0:64] = acc_ref[0, pl.ds((1 - zb0) * 64, 64)].astype(bf)
    stage_ref[2, 0:64] = acc_ref[1, pl.ds((1 - zb0) * 64, 64)].astype(bf)
    ex0R = z_rdma(0, stage_ref.at[0, pl.ds(0, 64)], commz_ref.at[0], zp1)
    ex0L = z_rdma(1, stage_ref.at[2, pl.ds(0, 64)], commz_ref.at[1], zp1)
    ex0R.start()
    ex0L.start()
    ex0R.wait_recv()
    ex0L.wait_recv()
    pend += [ex0R, ex0L]

    rs[5][0].wait_send()
    rs[5][1].wait_send()
    stage_ref[1, 0:32] = (
        acc_ref[0, pl.ds(sq, 32)] + commz_ref[0, pl.ds(qs, 32)].astype(f32)
    ).astype(bf)
    stage_ref[3, 0:32] = (
        acc_ref[1, pl.ds(sq, 32)] + commz_ref[1, pl.ds(qs, 32)].astype(f32)
    ).astype(bf)
    ex1R = z_rdma(2, stage_ref.at[1, pl.ds(0, 32)],
                  commz_ref.at[2, pl.ds(0, 32)], zp2)
    ex1L = z_rdma(3, stage_ref.at[3, pl.ds(0, 32)],
                  commz_ref.at[3, pl.ds(0, 32)], zp2)
    ex1R.start()
    ex1L.start()
    acc_ref[0, pl.ds(qq, 32)] = (
        acc_ref[0, pl.ds(qq, 32)] + commz_ref[0, pl.ds(qk, 32)].astype(f32)
    )
    acc_ref[1, pl.ds(qq, 32)] = (
        acc_ref[1, pl.ds(qq, 32)] + commz_ref[1, pl.ds(qk, 32)].astype(f32)
    )
    ex1R.wait_recv()
    ex1L.wait_recv()
    out_ref[oR, pl.ds(qq, 32)] = (
        acc_ref[0, pl.ds(qq, 32)] + commz_ref[2, pl.ds(0, 32)].astype(f32)
    ).astype(bf)
    out_ref[oL, pl.ds(HC + qq, 32)] = (
        acc_ref[1, pl.ds(qq, 32)] + commz_ref[3, pl.ds(0, 32)].astype(f32)
    ).astype(bf)
    pend += [ex1R, ex1L]

    ag2R = z_rdma(4, out_ref.at[oR, pl.ds(qq, 32)],
                  out_ref.at[oR, pl.ds(qq, 32)], zp2)
    ag2L = z_rdma(5, out_ref.at[oL, pl.ds(HC + qq, 32)],
                  out_ref.at[oL, pl.ds(HC + qq, 32)], zp2)
    ag2R.start()
    ag2L.start()
    ag2R.wait_recv()
    ag2L.wait_recv()
    pend += [ag2R, ag2L]

    ag3R = z_rdma(6, out_ref.at[oR, pl.ds(hh, 64)],
                  out_ref.at[oR, pl.ds(hh, 64)], zp1)
    ag3L = z_rdma(7, out_ref.at[oL, pl.ds(HC + hh, 64)],
                  out_ref.at[oL, pl.ds(HC + hh, 64)], zp1)
    ag3R.start()
    ag3L.start()
    ag3R.wait_recv()
    ag3L.wait_recv()
    pend += [ag3R, ag3L]

    ag = []
    for g in range(G - 1):
        t = (G - 1) + g
        sR = t % 2
        sL = 2 + t % 2
        s_idxR = (r + 1 - g) % G
        s_idxL = (r - 1 + g) % G
        pl.semaphore_wait(credit_sem.at[sR], 1)
        pl.semaphore_wait(credit_sem.at[sL], 1)
        if g >= 1:
            ag[g - 1][0].wait_recv()
            ag[g - 1][1].wait_recv()
            if g <= 5:
                credit_to((t - 1) % 2, left)
                credit_to(2 + (t - 1) % 2, right)
        agR = ring_rdma(sR, out_ref.at[s_idxR, pl.ds(0, HC)],
                        out_ref.at[s_idxR, pl.ds(0, HC)], right)
        agL = ring_rdma(sL, out_ref.at[s_idxL, pl.ds(HC, HC)],
                        out_ref.at[s_idxL, pl.ds(HC, HC)], left)
        agR.start()
        agL.start()
        ag.append((agR, agL))
    ag[6][0].wait_recv()
    ag[6][1].wait_recv()

    for r in pend:
        r.wait_send()
    for pair in ag:
        pair[0].wait_send()
        pair[1].wait_send()


def _ar1_body(p_ref, x0_ref, mod_ref, x1_ref, xm2_ref, out_ref, acc_ref,
              commg_ref, stage_ref, commz_ref, sendg_sem, recvg_sem,
              z_send_sem, z_recv_sem, credit_sem):
    _ar_core(p_ref, out_ref, acc_ref, commg_ref, stage_ref, commz_ref,
             sendg_sem, recvg_sem, z_send_sem, z_recv_sem, credit_sem)
    f32 = jnp.float32
    for c in range(G):
        b = c // (G // 2)
        x1 = x0_ref[c] + mod_ref[0, b][None, :] * out_ref[c].astype(f32)
        x1_ref[c] = x1
        mu = jnp.mean(x1, axis=-1, keepdims=True)
        d = x1 - mu
        var = jnp.mean(d * d, axis=-1, keepdims=True)
        xm2_ref[c] = (
            d * lax.rsqrt(var + 1e-5) * mod_ref[1, b][None, :]
            + mod_ref[2, b][None, :]
        ).astype(xm2_ref.dtype)


def _ar2_body(p_ref, x1_ref, gm_ref, y_ref, out_ref, acc_ref,
              commg_ref, stage_ref, commz_ref, sendg_sem, recvg_sem,
              z_send_sem, z_recv_sem, credit_sem):
    _ar_core(p_ref, out_ref, acc_ref, commg_ref, stage_ref, commz_ref,
             sendg_sem, recvg_sem, z_send_sem, z_recv_sem, credit_sem)
    f32 = jnp.float32
    for c in range(G):
        b = c // (G // 2)
        y_ref[c] = x1_ref[c] + gm_ref[b][None, :] * out_ref[c].astype(f32)


_AR_SCRATCH = [
    pltpu.VMEM((G, 2 * HC, 1024), jnp.bfloat16),
    pltpu.VMEM((2, HC, 1024), jnp.float32),
    pltpu.VMEM((4, HC, 1024), jnp.bfloat16),
    pltpu.VMEM((4, HC, 1024), jnp.bfloat16),
    pltpu.VMEM((4, 64, 1024), jnp.bfloat16),
    pltpu.SemaphoreType.DMA((4,)),
    pltpu.SemaphoreType.DMA((4,)),
    pltpu.SemaphoreType.DMA((8,)),
    pltpu.SemaphoreType.DMA((8,)),
    pltpu.SemaphoreType.REGULAR((4,)),
]
_VMEM_SPEC = pl.BlockSpec(memory_space=pltpu.VMEM)


def _allreduce_epilogue1(attn_partial, x0c, mods):
    _, c, D = attn_partial.shape
    return pl.pallas_call(
        _ar1_body,
        out_shape=(
            jax.ShapeDtypeStruct((G, c, D), jnp.float32),
            jax.ShapeDtypeStruct((G, c, D), jnp.bfloat16),
        ),
        in_specs=[_VMEM_SPEC] * 3,
        out_specs=(_VMEM_SPEC, _VMEM_SPEC),
        scratch_shapes=_AR_SCRATCH,
        compiler_params=pltpu.CompilerParams(collective_id=0),
    )(attn_partial, x0c, mods)


def _allreduce_epilogue2(ff_partial, x1c, gm):
    _, c, D = ff_partial.shape
    return pl.pallas_call(
        _ar2_body,
        out_shape=jax.ShapeDtypeStruct((G, c, D), jnp.float32),
        in_specs=[_VMEM_SPEC] * 3,
        out_specs=_VMEM_SPEC,
        scratch_shapes=_AR_SCRATCH,
        compiler_params=pltpu.CompilerParams(collective_id=1),
    )(ff_partial, x1c, gm)


def _attn_body(q_ref, k_ref, v_ref, o_ref):
    f32 = jnp.float32
    q = q_ref[0]
    k = k_ref[0]
    v = v_ref[0]
    s = lax.dot_general(
        q, k, (((1,), (1,)), ((), ())), preferred_element_type=f32
    ) * 0.08838834764831843
    m = jnp.max(s, axis=-1, keepdims=True)
    e = jnp.exp(s - m)
    l = jnp.sum(e, axis=-1, keepdims=True)
    o = lax.dot_general(
        e.astype(o_ref.dtype), v, (((1,), (0,)), ((), ())),
        preferred_element_type=f32,
    )
    o_ref[0] = (o / l).astype(o_ref.dtype)


def _flash_attention(Q, K, V, Dh):
    B, S, HD = Q.shape
    H = HD // Dh
    spec = pl.BlockSpec((1, S, Dh), lambda b, h: (b, 0, h))
    return pl.pallas_call(
        _attn_body,
        grid=(B, H),
        in_specs=[spec, spec, spec],
        out_specs=spec,
        out_shape=jax.ShapeDtypeStruct((B, S, HD), Q.dtype),
    )(Q, K, V)


def kernel(x, Wq, Wk, Wv, Wo, t_emb, W_mod, W_ff1, W_ff2):
    f32 = jnp.float32
    bf16 = jnp.bfloat16
    B, S, D = x.shape
    Dh = 128
    H = Wq.shape[1] // Dh

    mod = t_emb @ W_mod
    sa, sha, ga, sm, shm, gm = jnp.split(mod, 6, axis=-1)

    def ln(h):
        m = h.mean(axis=-1, keepdims=True)
        v = h.var(axis=-1, keepdims=True)
        return (h - m) * lax.rsqrt(v + 1e-5)

    x0 = x
    xm = (ln(x0) * (1.0 + sa[:, None, :]) + sha[:, None, :]).astype(bf16)

    O = _flash_attention(
        xm @ Wq.astype(bf16), xm @ Wk.astype(bf16), xm @ Wv.astype(bf16), Dh
    )
    attn_partial = (O @ Wo.astype(bf16)).astype(f32)

    rows = B * S
    c = rows // G
    mods = jnp.stack([ga, 1.0 + sm, shm])
    x1c, xm2 = _allreduce_epilogue1(
        attn_partial.reshape(G, c, D), x0.reshape(G, c, D), mods
    )

    h = (xm2.reshape(B * S, D) @ W_ff1.astype(bf16)).astype(f32)
    h = h * jax.nn.sigmoid(h)
    ff_partial = (h.astype(bf16) @ W_ff2.astype(bf16)).astype(f32)

    y = _allreduce_epilogue2(ff_partial.reshape(G, c, D), x1c, gm)
    return y.reshape(B, S, D)


# device time: 263691 ns/iter; 1.0393x vs baseline; 1.0127x over previous
import jax
import jax.numpy as jnp
from jax import lax
from jax.experimental import pallas as pl
from jax.experimental.pallas import tpu as pltpu

G = 8
HC = 128
LAST_T = 2 * (G - 1) - 1

_MESH = pl.DeviceIdType.MESH

_INV = (0, 7, 6, 1, 2, 5, 4, 3)
_NEXT = (3, 0, 1, 4, 7, 2, 5, 6)
_PREV = (1, 2, 5, 0, 3, 6, 7, 4)


def _lut(idx, table):
    v = jnp.int32(table[7])
    for i in range(6, -1, -1):
        v = jnp.where(idx == i, jnp.int32(table[i]), v)
    return v


def _ar_core(p_ref, out_ref, acc_ref, commg_ref, stage_ref, commz_ref,
             sendg_sem, recvg_sem, z_send_sem, z_recv_sem, credit_sem,
             epi=None):
    my = lax.axis_index("i")
    p = my % G
    z = my // G
    r = _lut(p, _INV)
    right = z * G + _lut(p, _NEXT)
    left = z * G + _lut(p, _PREV)
    zp1 = my ^ 8
    zp2 = my ^ 16

    barrier = pltpu.get_barrier_semaphore()
    for nbr in (left, right, zp1, zp2):
        pl.semaphore_signal(barrier, inc=1, device_id=(nbr,),
                            device_id_type=_MESH)
    pl.semaphore_wait(barrier, 4)

    bf = out_ref.dtype
    f32 = jnp.float32

    def ring_rdma(slot, src, dst, dev):
        return pltpu.make_async_remote_copy(
            src_ref=src, dst_ref=dst,
            send_sem=sendg_sem.at[slot], recv_sem=recvg_sem.at[slot],
            device_id=(dev,), device_id_type=_MESH,
        )

    def z_rdma(i, src, dst, dev):
        return pltpu.make_async_remote_copy(
            src_ref=src, dst_ref=dst,
            send_sem=z_send_sem.at[i], recv_sem=z_recv_sem.at[i],
            device_id=(dev,), device_id_type=_MESH,
        )

    def credit_to(slot, dev):
        pl.semaphore_signal(credit_sem.at[slot], inc=1, device_id=(dev,),
                            device_id_type=_MESH)

    rs = []
    for k in range(G - 1):
        sR = k % 2
        sL = 2 + k % 2
        s_idxR = (r - k) % G
        s_idxL = (r + k) % G
        if k >= 2:
            pl.semaphore_wait(credit_sem.at[sR], 1)
            pl.semaphore_wait(credit_sem.at[sL], 1)
            rs[k - 2][0].wait_send()
            rs[k - 2][1].wait_send()
        if k == 0:
            stage_ref[sR] = p_ref[s_idxR, 0:HC].astype(bf)
            stage_ref[sL] = p_ref[s_idxL, HC:2 * HC].astype(bf)
        else:
            pR = (k - 1) % 2
            pL = 2 + (k - 1) % 2
            stage_ref[sR] = (
                p_ref[s_idxR, 0:HC] + commg_ref[pR].astype(f32)
            ).astype(bf)
            stage_ref[sL] = (
                p_ref[s_idxL, HC:2 * HC] + commg_ref[pL].astype(f32)
            ).astype(bf)
            credit_to(pR, left)
            credit_to(pL, right)
        rdmaR = ring_rdma(sR, stage_ref.at[sR], commg_ref.at[sR], right)
        rdmaL = ring_rdma(sL, stage_ref.at[sL], commg_ref.at[sL], left)
        rdmaR.start()
        rdmaL.start()
        rs.append((rdmaR, rdmaL))
        rdmaR.wait_recv()
        rdmaL.wait_recv()

    oR = (r + 1) % G
    oL = (r - 1) % G
    acc_ref[0] = p_ref[oR, 0:HC] + commg_ref[0].astype(f32)
    acc_ref[1] = p_ref[oL, HC:2 * HC] + commg_ref[2].astype(f32)
    credit_to(0, left)
    credit_to(2, right)

    zb0 = z % 2
    zb1 = z // 2
    hh = zb0 * 64
    qq = hh + zb1 * 32
    sq = hh + (1 - zb1) * 32
    qk = zb1 * 32
    qs = (1 - zb1) * 32

    pend = []

    rs[6][0].wait_send()
    rs[6][1].wait_send()
    stage_ref[0, 0:64] = acc_ref[0, pl.ds((1 - zb0) * 64, 64)].astype(bf)
    stage_ref[2, 0:64] = acc_ref[1, pl.ds((1 - zb0) * 64, 64)].astype(bf)
    ex0R = z_rdma(0, stage_ref.at[0, pl.ds(0, 64)], commz_ref.at[0], zp1)
    ex0L = z_rdma(1, stage_ref.at[2, pl.ds(0, 64)], commz_ref.at[1], zp1)
    ex0R.start()
    ex0L.start()
    ex0R.wait_recv()
    ex0L.wait_recv()
    pend += [ex0R, ex0L]

    rs[5][0].wait_send()
    rs[5][1].wait_send()
    stage_ref[1, 0:32] = (
        acc_ref[0, pl.ds(sq, 32)] + commz_ref[0, pl.ds(qs, 32)].astype(f32)
    ).astype(bf)
    stage_ref[3, 0:32] = (
        acc_ref[1, pl.ds(sq, 32)] + commz_ref[1, pl.ds(qs, 32)].astype(f32)
    ).astype(bf)
    ex1R = z_rdma(2, stage_ref.at[1, pl.ds(0, 32)],
                  commz_ref.at[2, pl.ds(0, 32)], zp2)
    ex1L = z_rdma(3, stage_ref.at[3, pl.ds(0, 32)],
                  commz_ref.at[3, pl.ds(0, 32)], zp2)
    ex1R.start()
    ex1L.start()
    acc_ref[0, pl.ds(qq, 32)] = (
        acc_ref[0, pl.ds(qq, 32)] + commz_ref[0, pl.ds(qk, 32)].astype(f32)
    )
    acc_ref[1, pl.ds(qq, 32)] = (
        acc_ref[1, pl.ds(qq, 32)] + commz_ref[1, pl.ds(qk, 32)].astype(f32)
    )
    ex1R.wait_recv()
    ex1L.wait_recv()
    out_ref[oR, pl.ds(qq, 32)] = (
        acc_ref[0, pl.ds(qq, 32)] + commz_ref[2, pl.ds(0, 32)].astype(f32)
    ).astype(bf)
    out_ref[oL, pl.ds(HC + qq, 32)] = (
        acc_ref[1, pl.ds(qq, 32)] + commz_ref[3, pl.ds(0, 32)].astype(f32)
    ).astype(bf)
    pend += [ex1R, ex1L]

    ag2R = z_rdma(4, out_ref.at[oR, pl.ds(qq, 32)],
                  out_ref.at[oR, pl.ds(qq, 32)], zp2)
    ag2L = z_rdma(5, out_ref.at[oL, pl.ds(HC + qq, 32)],
                  out_ref.at[oL, pl.ds(HC + qq, 32)], zp2)
    ag2R.start()
    ag2L.start()
    ag2R.wait_recv()
    ag2L.wait_recv()
    pend += [ag2R, ag2L]

    ag3R = z_rdma(6, out_ref.at[oR, pl.ds(hh, 64)],
                  out_ref.at[oR, pl.ds(hh, 64)], zp1)
    ag3L = z_rdma(7, out_ref.at[oL, pl.ds(HC + hh, 64)],
                  out_ref.at[oL, pl.ds(HC + hh, 64)], zp1)
    ag3R.start()
    ag3L.start()
    ag3R.wait_recv()
    ag3L.wait_recv()
    pend += [ag3R, ag3L]

    ag = []
    for g in range(G - 1):
        t = (G - 1) + g
        sR = t % 2
        sL = 2 + t % 2
        s_idxR = (r + 1 - g) % G
        s_idxL = (r - 1 + g) % G
        pl.semaphore_wait(credit_sem.at[sR], 1)
        pl.semaphore_wait(credit_sem.at[sL], 1)
        if g >= 1:
            ag[g - 1][0].wait_recv()
            ag[g - 1][1].wait_recv()
            if g <= 5:
                credit_to((t - 1) % 2, left)
                credit_to(2 + (t - 1) % 2, right)
        agR = ring_rdma(sR, out_ref.at[s_idxR, pl.ds(0, HC)],
                        out_ref.at[s_idxR, pl.ds(0, HC)], right)
        agL = ring_rdma(sL, out_ref.at[s_idxL, pl.ds(HC, HC)],
                        out_ref.at[s_idxL, pl.ds(HC, HC)], left)
        agR.start()
        agL.start()
        ag.append((agR, agL))
        if epi is not None:
            if g == 0:
                epi(oR, 0)
                epi(oL, 1)
            else:
                epi((r - (g - 1)) % G, 0)
                epi((r + (g - 1)) % G, 1)
    ag[6][0].wait_recv()
    ag[6][1].wait_recv()
    if epi is not None:
        epi((r - 6) % G, 0)
        epi((r + 6) % G, 1)

    for r in pend:
        r.wait_send()
    for pair in ag:
        pair[0].wait_send()
        pair[1].wait_send()


def _ar1_body(p_ref, x0_ref, mod_ref, x1_ref, xm2_ref, out_ref, acc_ref,
              commg_ref, stage_ref, commz_ref, sendg_sem, recvg_sem,
              z_send_sem, z_recv_sem, credit_sem):
    f32 = jnp.float32

    def epi(idx, half):
        lo = half * HC
        b = idx // (G // 2)
        x1 = (
            x0_ref[idx, pl.ds(lo, HC)]
            + mod_ref[b][None, :] * out_ref[idx, pl.ds(lo, HC)].astype(f32)
        )
        x1_ref[idx, pl.ds(lo, HC)] = x1
        mu = jnp.mean(x1, axis=-1, keepdims=True)
        d = x1 - mu
        var = jnp.mean(d * d, axis=-1, keepdims=True)
        xm2_ref[idx, pl.ds(lo, HC)] = (
            d * lax.rsqrt(var + 1e-5) * mod_ref[2 + b][None, :]
            + mod_ref[4 + b][None, :]
        ).astype(xm2_ref.dtype)

    _ar_core(p_ref, out_ref, acc_ref, commg_ref, stage_ref, commz_ref,
             sendg_sem, recvg_sem, z_send_sem, z_recv_sem, credit_sem,
             epi=epi)


def _ar2_body(p_ref, x1_ref, gm_ref, y_ref, out_ref, acc_ref,
              commg_ref, stage_ref, commz_ref, sendg_sem, recvg_sem,
              z_send_sem, z_recv_sem, credit_sem):
    f32 = jnp.float32

    def epi(idx, half):
        lo = half * HC
        b = idx // (G // 2)
        y_ref[idx, pl.ds(lo, HC)] = (
            x1_ref[idx, pl.ds(lo, HC)]
            + gm_ref[b][None, :] * out_ref[idx, pl.ds(lo, HC)].astype(f32)
        )

    _ar_core(p_ref, out_ref, acc_ref, commg_ref, stage_ref, commz_ref,
             sendg_sem, recvg_sem, z_send_sem, z_recv_sem, credit_sem,
             epi=epi)


_AR_SCRATCH = [
    pltpu.VMEM((G, 2 * HC, 1024), jnp.bfloat16),
    pltpu.VMEM((2, HC, 1024), jnp.float32),
    pltpu.VMEM((4, HC, 1024), jnp.bfloat16),
    pltpu.VMEM((4, HC, 1024), jnp.bfloat16),
    pltpu.VMEM((4, 64, 1024), jnp.bfloat16),
    pltpu.SemaphoreType.DMA((4,)),
    pltpu.SemaphoreType.DMA((4,)),
    pltpu.SemaphoreType.DMA((8,)),
    pltpu.SemaphoreType.DMA((8,)),
    pltpu.SemaphoreType.REGULAR((4,)),
]
_VMEM_SPEC = pl.BlockSpec(memory_space=pltpu.VMEM)


def _allreduce_epilogue1(attn_partial, x0c, mods):
    _, c, D = attn_partial.shape
    return pl.pallas_call(
        _ar1_body,
        out_shape=(
            jax.ShapeDtypeStruct((G, c, D), jnp.float32),
            jax.ShapeDtypeStruct((G, c, D), jnp.bfloat16),
        ),
        in_specs=[_VMEM_SPEC] * 3,
        out_specs=(_VMEM_SPEC, _VMEM_SPEC),
        scratch_shapes=_AR_SCRATCH,
        compiler_params=pltpu.CompilerParams(collective_id=0),
    )(attn_partial, x0c, mods)


def _allreduce_epilogue2(ff_partial, x1c, gm):
    _, c, D = ff_partial.shape
    return pl.pallas_call(
        _ar2_body,
        out_shape=jax.ShapeDtypeStruct((G, c, D), jnp.float32),
        in_specs=[_VMEM_SPEC] * 3,
        out_specs=_VMEM_SPEC,
        scratch_shapes=_AR_SCRATCH,
        compiler_params=pltpu.CompilerParams(collective_id=1),
    )(ff_partial, x1c, gm)


def _attn_body(q_ref, k_ref, v_ref, o_ref):
    f32 = jnp.float32
    q = q_ref[0]
    k = k_ref[0]
    v = v_ref[0]
    s = lax.dot_general(
        q, k, (((1,), (1,)), ((), ())), preferred_element_type=f32
    ) * 0.08838834764831843
    m = jnp.max(s, axis=-1, keepdims=True)
    e = jnp.exp(s - m)
    l = jnp.sum(e, axis=-1, keepdims=True)
    o = lax.dot_general(
        e.astype(o_ref.dtype), v, (((1,), (0,)), ((), ())),
        preferred_element_type=f32,
    )
    o_ref[0] = (o / l).astype(o_ref.dtype)


def _flash_attention(Q, K, V, Dh):
    B, S, HD = Q.shape
    H = HD // Dh
    spec = pl.BlockSpec((1, S, Dh), lambda b, h: (b, 0, h))
    return pl.pallas_call(
        _attn_body,
        grid=(B, H),
        in_specs=[spec, spec, spec],
        out_specs=spec,
        out_shape=jax.ShapeDtypeStruct((B, S, HD), Q.dtype),
    )(Q, K, V)


def kernel(x, Wq, Wk, Wv, Wo, t_emb, W_mod, W_ff1, W_ff2):
    f32 = jnp.float32
    bf16 = jnp.bfloat16
    B, S, D = x.shape
    Dh = 128
    H = Wq.shape[1] // Dh

    mod = t_emb @ W_mod
    sa, sha, ga, sm, shm, gm = jnp.split(mod, 6, axis=-1)

    def ln(h):
        m = h.mean(axis=-1, keepdims=True)
        v = h.var(axis=-1, keepdims=True)
        return (h - m) * lax.rsqrt(v + 1e-5)

    x0 = x
    xm = (ln(x0) * (1.0 + sa[:, None, :]) + sha[:, None, :]).astype(bf16)

    O = _flash_attention(
        xm @ Wq.astype(bf16), xm @ Wk.astype(bf16), xm @ Wv.astype(bf16), Dh
    )
    attn_partial = (O @ Wo.astype(bf16)).astype(f32)

    rows = B * S
    c = rows // G
    mods = jnp.concatenate([ga, 1.0 + sm, shm], axis=0)
    x1c, xm2 = _allreduce_epilogue1(
        attn_partial.reshape(G, c, D), x0.reshape(G, c, D), mods
    )

    h = (xm2.reshape(B * S, D) @ W_ff1.astype(bf16)).astype(f32)
    h = h * jax.nn.sigmoid(h)
    ff_partial = (h.astype(bf16) @ W_ff2.astype(bf16)).astype(f32)

    y = _allreduce_epilogue2(ff_partial.reshape(G, c, D), x1c, gm)
    return y.reshape(B, S, D)
